# baseline (device time: 18033 ns/iter reference)
import jax
import jax.numpy as jnp
from jax import lax
from jax.experimental import pallas as pl
from jax.experimental.pallas import tpu as pltpu

N_DEV = 4
N_TOK = 512
D_IN = 256
D_OUT = 512
N_EXP = 16
EXP_PER_DEV = N_EXP // N_DEV
CAP = 25
ROWS = N_TOK // N_DEV


def kernel(x, router_W, route_idx, expert_W):
    del router_W

    def body(x_ref, idx_ref, w_ref, out_ref,
             acc_ref, send_ref, recv_ref, send_sems, recv_sems):
        my = lax.axis_index("i")
        left = lax.rem(my + N_DEV - 1, N_DEV)
        right = lax.rem(my + 1, N_DEV)

        barrier_sem = pltpu.get_barrier_semaphore()
        for nbr in (left, right):
            pl.semaphore_signal(
                barrier_sem, inc=1,
                device_id=(nbr,), device_id_type=pl.DeviceIdType.MESH,
            )
        pl.semaphore_wait(barrier_sem, 2)

        idx = idx_ref[:, :]
        eids = lax.broadcasted_iota(jnp.int32, (N_TOK, N_EXP), 1)
        onehot = (idx == eids).astype(jnp.float32)
        row = lax.broadcasted_iota(jnp.int32, (N_TOK, N_TOK), 0)
        col = lax.broadcasted_iota(jnp.int32, (N_TOK, N_TOK), 1)
        tri = (row >= col).astype(jnp.float32)
        cnt = jnp.dot(tri, onehot, preferred_element_type=jnp.float32)
        keep = onehot * (cnt <= CAP).astype(jnp.float32)
        sel_e = lax.broadcasted_iota(jnp.int32, (N_EXP, EXP_PER_DEV), 0)
        sel_l = lax.broadcasted_iota(jnp.int32, (N_EXP, EXP_PER_DEV), 1)
        sel = (sel_e == my * EXP_PER_DEV + sel_l).astype(jnp.float32)
        keep_my = jnp.dot(keep, sel, preferred_element_type=jnp.float32)

        xb = x_ref[:, :].astype(jnp.bfloat16)
        acc = jnp.zeros((N_TOK, D_OUT), jnp.float32)
        for l in range(EXP_PER_DEV):
            xm = xb * keep_my[:, l:l + 1].astype(jnp.bfloat16)
            acc = acc + jnp.dot(
                xm, w_ref[l].astype(jnp.bfloat16),
                preferred_element_type=jnp.float32,
            )
        acc_ref[:, :] = acc

        for t in range(N_DEV - 1):
            c = lax.rem(my + N_DEV - t - 1, N_DEV)
            chunk = acc_ref[pl.ds(c * ROWS, ROWS), :]
            if t == 0:
                send_val = chunk
            else:
                send_val = chunk + recv_ref[t - 1].astype(jnp.float32)
            send_ref[t] = send_val.astype(jnp.bfloat16)
            rdma = pltpu.make_async_remote_copy(
                src_ref=send_ref.at[t],
                dst_ref=recv_ref.at[t],
                send_sem=send_sems.at[t],
                recv_sem=recv_sems.at[t],
                device_id=(right,),
                device_id_type=pl.DeviceIdType.MESH,
            )
            rdma.start()
            rdma.wait()

        own = acc_ref[pl.ds(my * ROWS, ROWS), :]
        out_ref[:, :] = own + recv_ref[N_DEV - 2].astype(jnp.float32)

    return pl.pallas_call(
        body,
        out_shape=jax.ShapeDtypeStruct((ROWS, D_OUT), jnp.float32),
        in_specs=[
            pl.BlockSpec(memory_space=pltpu.VMEM),
            pl.BlockSpec(memory_space=pltpu.VMEM),
            pl.BlockSpec(memory_space=pltpu.VMEM),
        ],
        out_specs=pl.BlockSpec(memory_space=pltpu.VMEM),
        scratch_shapes=[
            pltpu.VMEM((N_TOK, D_OUT), jnp.float32),
            pltpu.VMEM((N_DEV - 1, ROWS, D_OUT), jnp.bfloat16),
            pltpu.VMEM((N_DEV - 1, ROWS, D_OUT), jnp.bfloat16),
            pltpu.SemaphoreType.DMA((N_DEV - 1,)),
            pltpu.SemaphoreType.DMA((N_DEV - 1,)),
        ],
        compiler_params=pltpu.CompilerParams(collective_id=0),
    )(x, route_idx, expert_W)


# device time: 12930 ns/iter; 1.3947x vs baseline; 1.3947x over previous
import jax
import jax.numpy as jnp
from jax import lax
from jax.experimental import pallas as pl
from jax.experimental.pallas import tpu as pltpu

N_DEV = 4
N_TOK = 512
D_IN = 256
D_OUT = 512
N_EXP = 16
EXP_PER_DEV = N_EXP // N_DEV
CAP = 25
ROWS = N_TOK // N_DEV


def kernel(x, router_W, route_idx, expert_W):
    del router_W

    def body(x_ref, idx_ref, w_ref, out_ref,
             xb_ref, keep_ref, send_ref, recv_ref, send_sems, recv_sems):
        my = lax.axis_index("i")

        barrier_sem = pltpu.get_barrier_semaphore()
        for k in range(1, N_DEV):
            pl.semaphore_signal(
                barrier_sem, inc=1,
                device_id=(lax.rem(my + k, N_DEV),),
                device_id_type=pl.DeviceIdType.MESH,
            )
        pl.semaphore_wait(barrier_sem, N_DEV - 1)

        idx = idx_ref[:, :]
        lids = (lax.broadcasted_iota(jnp.int32, (N_TOK, EXP_PER_DEV), 1)
                + my * EXP_PER_DEV)
        onehot = (idx == lids).astype(jnp.float32)
        row = lax.broadcasted_iota(jnp.int32, (N_TOK, N_TOK), 0)
        col = lax.broadcasted_iota(jnp.int32, (N_TOK, N_TOK), 1)
        tri = (row >= col).astype(jnp.float32)
        cnt = jnp.dot(tri, onehot, preferred_element_type=jnp.float32)
        keep = onehot * (cnt <= CAP).astype(jnp.float32)
        keep_ref[:, :] = keep.astype(jnp.bfloat16)
        xb_ref[:, :] = x_ref[:, :].astype(jnp.bfloat16)

        wbs = [w_ref[l].astype(jnp.bfloat16) for l in range(EXP_PER_DEV)]

        def chunk_out(c):
            xc = xb_ref[pl.ds(c * ROWS, ROWS), :]
            kc = keep_ref[pl.ds(c * ROWS, ROWS), :]
            acc = jnp.zeros((ROWS, D_OUT), jnp.float32)
            for l in range(EXP_PER_DEV):
                acc = acc + jnp.dot(
                    xc * kc[:, l:l + 1], wbs[l],
                    preferred_element_type=jnp.float32,
                )
            return acc

        descs = {}
        for k in (2, 1, 3):
            c = lax.rem(my + k, N_DEV)
            send_ref[k - 1] = chunk_out(c).astype(jnp.bfloat16)
            rdma = pltpu.make_async_remote_copy(
                src_ref=send_ref.at[k - 1],
                dst_ref=recv_ref.at[k - 1],
                send_sem=send_sems.at[k - 1],
                recv_sem=recv_sems.at[k - 1],
                device_id=(c,),
                device_id_type=pl.DeviceIdType.MESH,
            )
            rdma.start()
            descs[k] = rdma

        out_ref[:, :] = chunk_out(my)

        for k in (1, 3, 2):
            descs[k].wait_recv()
            out_ref[:, :] = out_ref[:, :] + recv_ref[k - 1].astype(jnp.float32)
        for k in (1, 2, 3):
            descs[k].wait_send()

    return pl.pallas_call(
        body,
        out_shape=jax.ShapeDtypeStruct((ROWS, D_OUT), jnp.float32),
        in_specs=[
            pl.BlockSpec(memory_space=pltpu.VMEM),
            pl.BlockSpec(memory_space=pltpu.VMEM),
            pl.BlockSpec(memory_space=pltpu.VMEM),
        ],
        out_specs=pl.BlockSpec(memory_space=pltpu.VMEM),
        scratch_shapes=[
            pltpu.VMEM((N_TOK, D_IN), jnp.bfloat16),
            pltpu.VMEM((N_TOK, EXP_PER_DEV), jnp.bfloat16),
            pltpu.VMEM((N_DEV - 1, ROWS, D_OUT), jnp.bfloat16),
            pltpu.VMEM((N_DEV - 1, ROWS, D_OUT), jnp.bfloat16),
            pltpu.SemaphoreType.DMA((N_DEV - 1,)),
            pltpu.SemaphoreType.DMA((N_DEV - 1,)),
        ],
        compiler_params=pltpu.CompilerParams(collective_id=0),
    )(x, route_idx, expert_W)


# device time: 12539 ns/iter; 1.4382x vs baseline; 1.0312x over previous
import jax
import jax.numpy as jnp
from jax import lax
from jax.experimental import pallas as pl
from jax.experimental.pallas import tpu as pltpu

N_DEV = 4
N_TOK = 512
D_IN = 256
D_OUT = 512
N_EXP = 16
EXP_PER_DEV = N_EXP // N_DEV
CAP = 25
ROWS = N_TOK // N_DEV


def kernel(x, router_W, route_idx, expert_W):
    del router_W

    tri = jnp.tril(jnp.ones((N_TOK, N_TOK), jnp.bfloat16))
    w_big = expert_W.reshape(EXP_PER_DEV * D_IN, D_OUT)

    def body(x_ref, idx_ref, tri_ref, w_ref, out_ref,
             xbig_ref, send_ref, recv_ref, send_sems, recv_sems):
        me = lax.axis_index("i")

        barrier_sem = pltpu.get_barrier_semaphore()
        for k in range(1, N_DEV):
            pl.semaphore_signal(
                barrier_sem, inc=1,
                device_id=(lax.rem(me + k, N_DEV),),
                device_id_type=pl.DeviceIdType.MESH,
            )

        idx = idx_ref[:, :]
        lids = (lax.broadcasted_iota(jnp.int32, (N_TOK, EXP_PER_DEV), 1)
                + me * EXP_PER_DEV)
        onehot = (idx == lids).astype(jnp.bfloat16)
        cnt = jnp.dot(tri_ref[:, :], onehot, preferred_element_type=jnp.float32)
        kp = onehot * (cnt <= CAP).astype(jnp.bfloat16)

        xb = x_ref[:, :].astype(jnp.bfloat16)
        for l in range(EXP_PER_DEV):
            xbig_ref[:, l * D_IN:(l + 1) * D_IN] = xb * kp[:, l:l + 1]

        wb = w_ref[:, :].astype(jnp.bfloat16)

        def chunk_gemm(c, prefer_f32=False):
            xmc = xbig_ref[pl.ds(c * ROWS, ROWS), :]
            if prefer_f32:
                return jnp.dot(xmc, wb, preferred_element_type=jnp.float32)
            return jnp.dot(
                xmc, wb, preferred_element_type=jnp.float32
            ).astype(jnp.bfloat16)

        first_k = 2
        send_ref[first_k - 1] = chunk_gemm(lax.rem(me + first_k, N_DEV))

        pl.semaphore_wait(barrier_sem, N_DEV - 1)

        descs = {}
        for k in (2, 1, 3):
            c = lax.rem(me + k, N_DEV)
            if k != first_k:
                send_ref[k - 1] = chunk_gemm(c)
            rdma = pltpu.make_async_remote_copy(
                src_ref=send_ref.at[k - 1],
                dst_ref=recv_ref.at[k - 1],
                send_sem=send_sems.at[k - 1],
                recv_sem=recv_sems.at[k - 1],
                device_id=(c,),
                device_id_type=pl.DeviceIdType.MESH,
            )
            rdma.start()
            descs[k] = rdma

        out_ref[:, :] = chunk_gemm(me, prefer_f32=True)

        for k in (1, 3, 2):
            descs[k].wait_recv()
            out_ref[:, :] = out_ref[:, :] + recv_ref[k - 1].astype(jnp.float32)
        for k in (1, 2, 3):
            descs[k].wait_send()

    return pl.pallas_call(
        body,
        out_shape=jax.ShapeDtypeStruct((ROWS, D_OUT), jnp.float32),
        in_specs=[
            pl.BlockSpec(memory_space=pltpu.VMEM),
            pl.BlockSpec(memory_space=pltpu.VMEM),
            pl.BlockSpec(memory_space=pltpu.VMEM),
            pl.BlockSpec(memory_space=pltpu.VMEM),
        ],
        out_specs=pl.BlockSpec(memory_space=pltpu.VMEM),
        scratch_shapes=[
            pltpu.VMEM((N_TOK, EXP_PER_DEV * D_IN), jnp.bfloat16),
            pltpu.VMEM((N_DEV - 1, ROWS, D_OUT), jnp.bfloat16),
            pltpu.VMEM((N_DEV - 1, ROWS, D_OUT), jnp.bfloat16),
            pltpu.SemaphoreType.DMA((N_DEV - 1,)),
            pltpu.SemaphoreType.DMA((N_DEV - 1,)),
        ],
        compiler_params=pltpu.CompilerParams(collective_id=0),
    )(x, route_idx, tri, w_big)
